# Initial kernel scaffold; baseline (speedup 1.0000x reference)
import jax
import jax.numpy as jnp
from jax.experimental import pallas as pl


def kernel(x, edge_index, W1, b1, W2, b2, lin1_W, lin1_b, lin3_W, lin3_b):
    raise NotImplementedError("write your pallas kernel here")


# speedup vs baseline: 15.6665x; 15.6665x over previous
import functools

import jax
import jax.numpy as jnp
from jax import lax
from jax.experimental import pallas as pl
from jax.experimental.pallas import tpu as pltpu
from jax.experimental.pallas import tpu_sc as plsc

N_NODES = 1024
N_EDGES = 65536
IN_DIM = 4
HID = 8
LANES = 16
N_SUB = 16
NODES_PER_TILE = N_NODES // N_SUB
CHUNK = 128
ROWS_PER_TILE = (N_EDGES // CHUNK) // N_SUB


def _rsqrt16(d):
    i = plsc.bitcast(d, jnp.int32)
    i = 0x5F3759DF - lax.shift_right_logical(i, 1)
    y = plsc.bitcast(i, jnp.float32)
    for _ in range(3):
        y = y * (1.5 - 0.5 * d * y * y)
    return y


def _gcn_body(src_hbm, dst_hbm, x_hbm, w1_hbm, b1_hbm, w2_hbm, b2_hbm,
              ones_hbm, out_hbm,
              srcl, dstl, rows, onesv, xloc, w1v, b1v, w2v, b2v,
              degb, dinvb, xwb, hwb, accb, pubb, zb, outb,
              t_src, t_acc, sem):
    c = lax.axis_index("c")
    s = lax.axis_index("s")
    nd = pl.ds(s * NODES_PER_TILE, NODES_PER_TILE)

    pltpu.sync_copy(src_hbm.at[pl.ds(s * ROWS_PER_TILE, ROWS_PER_TILE)], srcl)
    pltpu.sync_copy(dst_hbm.at[pl.ds(s * ROWS_PER_TILE, ROWS_PER_TILE)], dstl)
    pltpu.sync_copy(ones_hbm, onesv)
    pltpu.sync_copy(x_hbm.at[nd], xloc)
    pltpu.sync_copy(w1_hbm, w1v)
    pltpu.sync_copy(b1_hbm, b1v)
    pltpu.sync_copy(w2_hbm, w2v)
    pltpu.sync_copy(b2_hbm, b2v)
    pltpu.sync_copy(onesv.at[pl.ds(0, NODES_PER_TILE)], t_acc.at[nd])
    plsc.subcore_barrier()

    def deg_step(j, carry):
        pltpu.sync_copy(onesv, t_acc.at[dstl.at[j]], add=True)
        return carry
    lax.fori_loop(0, ROWS_PER_TILE, deg_step, 0)
    plsc.subcore_barrier()

    pltpu.sync_copy(t_acc.at[nd], degb)
    b1 = b1v[...]
    b2 = b2v[...]

    def loc1(i, carry):
        y = _rsqrt16(degb[i])
        dinvb[i] = y
        xrow = xloc[i]
        acc = jnp.zeros((LANES,), jnp.float32)
        for k in range(IN_DIM):
            acc = acc + w1v[k] * xrow[k]
        xwb[i] = acc
        pubb[i] = acc * y
        zb[i] = jnp.zeros((LANES,), jnp.float32)
        return carry
    lax.fori_loop(0, NODES_PER_TILE, loc1, 0)
    pltpu.sync_copy(pubb, t_src.at[nd])
    pltpu.sync_copy(zb, t_acc.at[nd])
    plsc.subcore_barrier()

    def edge_step(j, carry):
        pltpu.async_copy(t_src.at[srcl.at[j]], rows, sem).wait()
        pltpu.sync_copy(rows, t_acc.at[dstl.at[j]], add=True)
        return carry
    lax.fori_loop(0, ROWS_PER_TILE, edge_step, 0)
    plsc.subcore_barrier()

    pltpu.sync_copy(t_acc.at[nd], accb)

    def loc2(i, carry):
        y = dinvb[i]
        h1 = jnp.maximum(y * accb[i] + (y * y) * xwb[i] + b1, 0.0)
        accb[i] = h1
        return carry
    lax.fori_loop(0, NODES_PER_TILE, loc2, 0)

    def loc3(i, carry):
        h1row = accb[i]
        acc = jnp.zeros((LANES,), jnp.float32)
        for k in range(HID):
            acc = acc + w2v[k] * h1row[k]
        hwb[i] = acc
        pubb[i] = acc * dinvb[i]
        return carry
    lax.fori_loop(0, NODES_PER_TILE, loc3, 0)
    pltpu.sync_copy(pubb, t_src.at[nd])
    pltpu.sync_copy(zb, t_acc.at[nd])
    plsc.subcore_barrier()

    lax.fori_loop(0, ROWS_PER_TILE, edge_step, 0)
    plsc.subcore_barrier()

    pltpu.sync_copy(t_acc.at[nd], accb)

    def loc4(i, carry):
        y = dinvb[i]
        outb[i] = y * accb[i] + (y * y) * hwb[i] + b2
        return carry
    lax.fori_loop(0, NODES_PER_TILE, loc4, 0)

    @pl.when(c == 0)
    def _():
        pltpu.sync_copy(outb, out_hbm.at[nd])


def _gcn_sc(src2, dst2, x, w1p, b1p, w2p, b2p, ones):
    mesh = plsc.VectorSubcoreMesh(core_axis_name="c", subcore_axis_name="s")
    f32 = jnp.float32
    kern = pl.kernel(
        _gcn_body,
        out_type=jax.ShapeDtypeStruct((N_NODES, LANES), f32),
        mesh=mesh,
        compiler_params=pltpu.CompilerParams(needs_layout_passes=False),
        scratch_types=[
            pltpu.VMEM((ROWS_PER_TILE, CHUNK), jnp.int32),
            pltpu.VMEM((ROWS_PER_TILE, CHUNK), jnp.int32),
            pltpu.VMEM((CHUNK, LANES), f32),
            pltpu.VMEM((CHUNK, LANES), f32),
            pltpu.VMEM((NODES_PER_TILE, LANES), f32),
            pltpu.VMEM((IN_DIM, LANES), f32),
            pltpu.VMEM((LANES,), f32),
            pltpu.VMEM((HID, LANES), f32),
            pltpu.VMEM((LANES,), f32),
            pltpu.VMEM((NODES_PER_TILE, LANES), f32),
            pltpu.VMEM((NODES_PER_TILE, LANES), f32),
            pltpu.VMEM((NODES_PER_TILE, LANES), f32),
            pltpu.VMEM((NODES_PER_TILE, LANES), f32),
            pltpu.VMEM((NODES_PER_TILE, LANES), f32),
            pltpu.VMEM((NODES_PER_TILE, LANES), f32),
            pltpu.VMEM((NODES_PER_TILE, LANES), f32),
            pltpu.VMEM((NODES_PER_TILE, LANES), f32),
            pltpu.VMEM_SHARED((N_NODES, LANES), f32),
            pltpu.VMEM_SHARED((N_NODES, LANES), f32),
            pltpu.SemaphoreType.DMA,
        ],
    )
    return kern(src2, dst2, x, w1p, b1p, w2p, b2p, ones)


BLK = 256


def _mlp_body(v_ref, w1_ref, b1_ref, w3_ref, b3_ref, o_ref, acc_ref):
    i = pl.program_id(0)
    u = jnp.dot(v_ref[...], w1_ref[...], preferred_element_type=jnp.float32)
    u = jnp.maximum(u + b1_ref[...], 0.0)
    p = jnp.dot(u, w3_ref[...], preferred_element_type=jnp.float32)

    @pl.when(i == 0)
    def _():
        acc_ref[...] = p

    @pl.when(i > 0)
    def _():
        acc_ref[...] = acc_ref[...] + p

    @pl.when(i == pl.num_programs(0) - 1)
    def _():
        z = acc_ref[...] + b3_ref[...]
        z = z - jnp.max(z, axis=-1, keepdims=True)
        e = jnp.exp(z)
        o_ref[...] = e / jnp.sum(e, axis=-1, keepdims=True)


def _mlp_tc(v, lin1_W, lin1_b, lin3_W, lin3_b):
    n_hidden = lin1_W.shape[1]
    n_out = lin3_W.shape[1]
    return pl.pallas_call(
        _mlp_body,
        grid=(n_hidden // BLK,),
        in_specs=[
            pl.BlockSpec((1, v.shape[1]), lambda i: (0, 0)),
            pl.BlockSpec((v.shape[1], BLK), lambda i: (0, i)),
            pl.BlockSpec((1, BLK), lambda i: (0, i)),
            pl.BlockSpec((BLK, n_out), lambda i: (i, 0)),
            pl.BlockSpec((1, n_out), lambda i: (0, 0)),
        ],
        out_specs=pl.BlockSpec((1, n_out), lambda i: (0, 0)),
        out_shape=jax.ShapeDtypeStruct((1, n_out), jnp.float32),
        scratch_shapes=[pltpu.VMEM((1, n_out), jnp.float32)],
    )(v, lin1_W, lin1_b, lin3_W, lin3_b)


def kernel(x, edge_index, W1, b1, W2, b2, lin1_W, lin1_b, lin3_W, lin3_b):
    f32 = jnp.float32
    src2 = edge_index[0].reshape(N_EDGES // CHUNK, CHUNK)
    dst2 = edge_index[1].reshape(N_EDGES // CHUNK, CHUNK)
    x16 = jnp.zeros((N_NODES, LANES), f32).at[:, :IN_DIM].set(x)
    w1p = jnp.zeros((IN_DIM, LANES), f32).at[:, :HID].set(W1)
    b1p = jnp.zeros((LANES,), f32).at[:HID].set(b1)
    w2p = jnp.zeros((HID, LANES), f32).at[:, :HID].set(W2)
    b2p = jnp.zeros((LANES,), f32).at[:HID].set(b2)
    ones = jnp.ones((CHUNK, LANES), f32)
    h2 = _gcn_sc(src2, dst2, x16, w1p, b1p, w2p, b2p, ones)
    v = h2[:, :HID].reshape(1, -1)
    out = _mlp_tc(v, lin1_W, lin1_b.reshape(1, -1), lin3_W, lin3_b.reshape(1, -1))
    return out.reshape(-1)


# speedup vs baseline: 16.1878x; 1.0333x over previous
import functools

import jax
import jax.numpy as jnp
from jax import lax
from jax.experimental import pallas as pl
from jax.experimental.pallas import tpu as pltpu
from jax.experimental.pallas import tpu_sc as plsc

N_NODES = 1024
N_EDGES = 65536
IN_DIM = 4
HID = 8
LANES = 16
N_SUB = 16
NODES_PER_TILE = N_NODES // N_SUB
CHUNK = 128
ROWS_PER_TILE = (N_EDGES // CHUNK) // N_SUB


def _rsqrt16(d):
    i = plsc.bitcast(d, jnp.int32)
    i = 0x5F3759DF - lax.shift_right_logical(i, 1)
    y = plsc.bitcast(i, jnp.float32)
    for _ in range(3):
        y = y * (1.5 - 0.5 * d * y * y)
    return y


def _gcn_body(src_hbm, dst_hbm, x_hbm, w1_hbm, b1_hbm, w2_hbm, b2_hbm,
              ones_hbm, out_hbm,
              srcl, dstl, rows, onesv, xloc, w1v, b1v, w2v, b2v,
              degb, dinvb, xwb, hwb, accb, pubb, zb, outb,
              t_src, t_acc, sem):
    c = lax.axis_index("c")
    s = lax.axis_index("s")
    nd = pl.ds(s * NODES_PER_TILE, NODES_PER_TILE)

    pltpu.sync_copy(src_hbm.at[pl.ds(s * ROWS_PER_TILE, ROWS_PER_TILE)], srcl)
    pltpu.sync_copy(dst_hbm.at[pl.ds(s * ROWS_PER_TILE, ROWS_PER_TILE)], dstl)
    pltpu.sync_copy(ones_hbm, onesv)
    pltpu.sync_copy(x_hbm.at[nd], xloc)
    pltpu.sync_copy(w1_hbm, w1v)
    pltpu.sync_copy(b1_hbm, b1v)
    pltpu.sync_copy(w2_hbm, w2v)
    pltpu.sync_copy(b2_hbm, b2v)
    pltpu.sync_copy(onesv.at[pl.ds(0, NODES_PER_TILE)], t_acc.at[nd])
    plsc.subcore_barrier()

    def deg_step(j, carry):
        pltpu.sync_copy(onesv, t_acc.at[dstl.at[j]], add=True)
        return carry
    lax.fori_loop(0, ROWS_PER_TILE, deg_step, 0)
    plsc.subcore_barrier()

    pltpu.sync_copy(t_acc.at[nd], degb)
    b1 = b1v[...]
    b2 = b2v[...]

    def loc1(i, carry):
        y = _rsqrt16(degb[i])
        dinvb[i] = y
        xrow = xloc[i]
        acc = jnp.zeros((LANES,), jnp.float32)
        for k in range(IN_DIM):
            acc = acc + w1v[k] * xrow[k]
        xwb[i] = acc
        pubb[i] = acc * y
        zb[i] = jnp.zeros((LANES,), jnp.float32)
        return carry
    lax.fori_loop(0, NODES_PER_TILE, loc1, 0)
    pltpu.sync_copy(pubb, t_src.at[nd])
    pltpu.sync_copy(zb, t_acc.at[nd])
    plsc.subcore_barrier()

    def edge_step(j, carry):
        pltpu.async_copy(t_src.at[srcl.at[j]], rows, sem).wait()
        pltpu.sync_copy(rows, t_acc.at[dstl.at[j]], add=True)
        return carry
    lax.fori_loop(0, ROWS_PER_TILE, edge_step, 0)
    plsc.subcore_barrier()

    pltpu.sync_copy(t_acc.at[nd], accb)

    def loc2(i, carry):
        y = dinvb[i]
        h1 = jnp.maximum(y * accb[i] + (y * y) * xwb[i] + b1, 0.0)
        accb[i] = h1
        return carry
    lax.fori_loop(0, NODES_PER_TILE, loc2, 0)

    def loc3(i, carry):
        h1row = accb[i]
        acc = jnp.zeros((LANES,), jnp.float32)
        for k in range(HID):
            acc = acc + w2v[k] * h1row[k]
        hwb[i] = acc
        pubb[i] = acc * dinvb[i]
        return carry
    lax.fori_loop(0, NODES_PER_TILE, loc3, 0)
    pltpu.sync_copy(pubb, t_src.at[nd])
    pltpu.sync_copy(zb, t_acc.at[nd])
    plsc.subcore_barrier()

    lax.fori_loop(0, ROWS_PER_TILE, edge_step, 0)
    plsc.subcore_barrier()

    pltpu.sync_copy(t_acc.at[nd], accb)

    def loc4(i, carry):
        y = dinvb[i]
        outb[i] = y * accb[i] + (y * y) * hwb[i] + b2
        return carry
    lax.fori_loop(0, NODES_PER_TILE, loc4, 0)

    @pl.when(c == 0)
    def _():
        pltpu.sync_copy(outb, out_hbm.at[nd])


def _gcn_sc(src2, dst2, x, w1p, b1p, w2p, b2p, ones):
    mesh = plsc.VectorSubcoreMesh(core_axis_name="c", subcore_axis_name="s",
                                  num_cores=1)
    f32 = jnp.float32
    kern = pl.kernel(
        _gcn_body,
        out_type=jax.ShapeDtypeStruct((N_NODES, LANES), f32),
        mesh=mesh,
        compiler_params=pltpu.CompilerParams(needs_layout_passes=False),
        scratch_types=[
            pltpu.VMEM((ROWS_PER_TILE, CHUNK), jnp.int32),
            pltpu.VMEM((ROWS_PER_TILE, CHUNK), jnp.int32),
            pltpu.VMEM((CHUNK, LANES), f32),
            pltpu.VMEM((CHUNK, LANES), f32),
            pltpu.VMEM((NODES_PER_TILE, LANES), f32),
            pltpu.VMEM((IN_DIM, LANES), f32),
            pltpu.VMEM((LANES,), f32),
            pltpu.VMEM((HID, LANES), f32),
            pltpu.VMEM((LANES,), f32),
            pltpu.VMEM((NODES_PER_TILE, LANES), f32),
            pltpu.VMEM((NODES_PER_TILE, LANES), f32),
            pltpu.VMEM((NODES_PER_TILE, LANES), f32),
            pltpu.VMEM((NODES_PER_TILE, LANES), f32),
            pltpu.VMEM((NODES_PER_TILE, LANES), f32),
            pltpu.VMEM((NODES_PER_TILE, LANES), f32),
            pltpu.VMEM((NODES_PER_TILE, LANES), f32),
            pltpu.VMEM((NODES_PER_TILE, LANES), f32),
            pltpu.VMEM_SHARED((N_NODES, LANES), f32),
            pltpu.VMEM_SHARED((N_NODES, LANES), f32),
            pltpu.SemaphoreType.DMA,
        ],
    )
    return kern(src2, dst2, x, w1p, b1p, w2p, b2p, ones)


BLK = 256


def _mlp_body(v_ref, w1_ref, b1_ref, w3_ref, b3_ref, o_ref, acc_ref):
    i = pl.program_id(0)
    u = jnp.dot(v_ref[...], w1_ref[...], preferred_element_type=jnp.float32)
    u = jnp.maximum(u + b1_ref[...], 0.0)
    p = jnp.dot(u, w3_ref[...], preferred_element_type=jnp.float32)

    @pl.when(i == 0)
    def _():
        acc_ref[...] = p

    @pl.when(i > 0)
    def _():
        acc_ref[...] = acc_ref[...] + p

    @pl.when(i == pl.num_programs(0) - 1)
    def _():
        z = acc_ref[...] + b3_ref[...]
        z = z - jnp.max(z, axis=-1, keepdims=True)
        e = jnp.exp(z)
        o_ref[...] = e / jnp.sum(e, axis=-1, keepdims=True)


def _mlp_tc(v, lin1_W, lin1_b, lin3_W, lin3_b):
    n_hidden = lin1_W.shape[1]
    n_out = lin3_W.shape[1]
    return pl.pallas_call(
        _mlp_body,
        grid=(n_hidden // BLK,),
        in_specs=[
            pl.BlockSpec((1, v.shape[1]), lambda i: (0, 0)),
            pl.BlockSpec((v.shape[1], BLK), lambda i: (0, i)),
            pl.BlockSpec((1, BLK), lambda i: (0, i)),
            pl.BlockSpec((BLK, n_out), lambda i: (i, 0)),
            pl.BlockSpec((1, n_out), lambda i: (0, 0)),
        ],
        out_specs=pl.BlockSpec((1, n_out), lambda i: (0, 0)),
        out_shape=jax.ShapeDtypeStruct((1, n_out), jnp.float32),
        scratch_shapes=[pltpu.VMEM((1, n_out), jnp.float32)],
    )(v, lin1_W, lin1_b, lin3_W, lin3_b)


def kernel(x, edge_index, W1, b1, W2, b2, lin1_W, lin1_b, lin3_W, lin3_b):
    f32 = jnp.float32
    src2 = edge_index[0].reshape(N_EDGES // CHUNK, CHUNK)
    dst2 = edge_index[1].reshape(N_EDGES // CHUNK, CHUNK)
    x16 = jnp.zeros((N_NODES, LANES), f32).at[:, :IN_DIM].set(x)
    w1p = jnp.zeros((IN_DIM, LANES), f32).at[:, :HID].set(W1)
    b1p = jnp.zeros((LANES,), f32).at[:HID].set(b1)
    w2p = jnp.zeros((HID, LANES), f32).at[:, :HID].set(W2)
    b2p = jnp.zeros((LANES,), f32).at[:HID].set(b2)
    ones = jnp.ones((CHUNK, LANES), f32)
    h2 = _gcn_sc(src2, dst2, x16, w1p, b1p, w2p, b2p, ones)
    v = h2[:, :HID].reshape(1, -1)
    out = _mlp_tc(v, lin1_W, lin1_b.reshape(1, -1), lin3_W, lin3_b.reshape(1, -1))
    return out.reshape(-1)


# speedup vs baseline: 19.1509x; 1.1830x over previous
import functools

import jax
import jax.numpy as jnp
from jax import lax
from jax.experimental import pallas as pl
from jax.experimental.pallas import tpu as pltpu
from jax.experimental.pallas import tpu_sc as plsc

N_NODES = 1024
N_EDGES = 65536
IN_DIM = 4
HID = 8
LANES = 16
N_SUB = 16
NODES_PER_TILE = N_NODES // N_SUB
CHUNK = 128
ROWS_PER_TILE = (N_EDGES // CHUNK) // N_SUB


def _rsqrt16(d):
    i = plsc.bitcast(d, jnp.int32)
    i = 0x5F3759DF - lax.shift_right_logical(i, 1)
    y = plsc.bitcast(i, jnp.float32)
    for _ in range(3):
        y = y * (1.5 - 0.5 * d * y * y)
    return y


NBUF = 8
N_ROUNDS = ROWS_PER_TILE // NBUF


def _gcn_body(src_hbm, dst_hbm, x_hbm, w1_hbm, b1_hbm, w2_hbm, b2_hbm,
              ones_hbm, out_hbm,
              srcl, dstl, rowsb, onesv, xloc, w1v, b1v, w2v, b2v,
              degb, dinvb, xwb, hwb, accb, pubb, zb, outb,
              t_src, t_acc, sem_g, sem_s):
    c = lax.axis_index("c")
    s = lax.axis_index("s")
    nd = pl.ds(s * NODES_PER_TILE, NODES_PER_TILE)

    stage = [
        pltpu.async_copy(src_hbm.at[pl.ds(s * ROWS_PER_TILE, ROWS_PER_TILE)],
                         srcl, sem_g),
        pltpu.async_copy(dst_hbm.at[pl.ds(s * ROWS_PER_TILE, ROWS_PER_TILE)],
                         dstl, sem_g),
        pltpu.async_copy(ones_hbm, onesv, sem_g),
        pltpu.async_copy(x_hbm.at[nd], xloc, sem_g),
        pltpu.async_copy(w1_hbm, w1v, sem_g),
        pltpu.async_copy(b1_hbm, b1v, sem_g),
        pltpu.async_copy(w2_hbm, w2v, sem_g),
        pltpu.async_copy(b2_hbm, b2v, sem_g),
    ]
    for d in stage:
        d.wait()
    pltpu.sync_copy(onesv.at[pl.ds(0, NODES_PER_TILE)], t_acc.at[nd])
    plsc.subcore_barrier()

    deg_ds = [pltpu.async_copy(onesv, t_acc.at[dstl.at[j]], sem_s, add=True)
              for j in range(ROWS_PER_TILE)]

    def mac1(i, carry):
        xrow = xloc[i]
        acc = jnp.zeros((LANES,), jnp.float32)
        for k in range(IN_DIM):
            acc = acc + w1v[k] * xrow[k]
        xwb[i] = acc
        zb[i] = jnp.zeros((LANES,), jnp.float32)
        return carry
    lax.fori_loop(0, NODES_PER_TILE, mac1, 0)
    for d in deg_ds:
        d.wait()
    plsc.subcore_barrier()

    pltpu.sync_copy(t_acc.at[nd], degb)
    b1 = b1v[...]
    b2 = b2v[...]

    def loc1(i, carry):
        y = _rsqrt16(degb[i])
        dinvb[i] = y
        pubb[i] = xwb[i] * y
        return carry
    lax.fori_loop(0, NODES_PER_TILE, loc1, 0)
    pltpu.sync_copy(pubb, t_src.at[nd])
    pltpu.sync_copy(zb, t_acc.at[nd])
    plsc.subcore_barrier()

    def edge_pass():
        for r in range(N_ROUNDS):
            gs = [pltpu.async_copy(t_src.at[srcl.at[r * NBUF + b]],
                                   rowsb.at[b], sem_g)
                  for b in range(NBUF)]
            for d in gs:
                d.wait()
            ss = [pltpu.async_copy(rowsb.at[b],
                                   t_acc.at[dstl.at[r * NBUF + b]],
                                   sem_s, add=True)
                  for b in range(NBUF)]
            for d in ss:
                d.wait()

    edge_pass()
    plsc.subcore_barrier()

    pltpu.sync_copy(t_acc.at[nd], accb)

    def loc2(i, carry):
        y = dinvb[i]
        h1 = jnp.maximum(y * accb[i] + (y * y) * xwb[i] + b1, 0.0)
        accb[i] = h1
        return carry
    lax.fori_loop(0, NODES_PER_TILE, loc2, 0)

    def loc3(i, carry):
        h1row = accb[i]
        acc = jnp.zeros((LANES,), jnp.float32)
        for k in range(HID):
            acc = acc + w2v[k] * h1row[k]
        hwb[i] = acc
        pubb[i] = acc * dinvb[i]
        return carry
    lax.fori_loop(0, NODES_PER_TILE, loc3, 0)
    pltpu.sync_copy(pubb, t_src.at[nd])
    pltpu.sync_copy(zb, t_acc.at[nd])
    plsc.subcore_barrier()

    edge_pass()
    plsc.subcore_barrier()

    pltpu.sync_copy(t_acc.at[nd], accb)

    def loc4(i, carry):
        y = dinvb[i]
        outb[i] = y * accb[i] + (y * y) * hwb[i] + b2
        return carry
    lax.fori_loop(0, NODES_PER_TILE, loc4, 0)

    @pl.when(c == 0)
    def _():
        pltpu.sync_copy(outb, out_hbm.at[nd])


def _gcn_sc(src2, dst2, x, w1p, b1p, w2p, b2p, ones):
    mesh = plsc.VectorSubcoreMesh(core_axis_name="c", subcore_axis_name="s",
                                  num_cores=1)
    f32 = jnp.float32
    kern = pl.kernel(
        _gcn_body,
        out_type=jax.ShapeDtypeStruct((N_NODES, LANES), f32),
        mesh=mesh,
        compiler_params=pltpu.CompilerParams(needs_layout_passes=False,
                                             use_tc_tiling_on_sc=False),
        scratch_types=[
            pltpu.VMEM((ROWS_PER_TILE, CHUNK), jnp.int32),
            pltpu.VMEM((ROWS_PER_TILE, CHUNK), jnp.int32),
            pltpu.VMEM((NBUF, CHUNK, LANES), f32),
            pltpu.VMEM((CHUNK, LANES), f32),
            pltpu.VMEM((NODES_PER_TILE, LANES), f32),
            pltpu.VMEM((IN_DIM, LANES), f32),
            pltpu.VMEM((LANES,), f32),
            pltpu.VMEM((HID, LANES), f32),
            pltpu.VMEM((LANES,), f32),
            pltpu.VMEM((NODES_PER_TILE, LANES), f32),
            pltpu.VMEM((NODES_PER_TILE, LANES), f32),
            pltpu.VMEM((NODES_PER_TILE, LANES), f32),
            pltpu.VMEM((NODES_PER_TILE, LANES), f32),
            pltpu.VMEM((NODES_PER_TILE, LANES), f32),
            pltpu.VMEM((NODES_PER_TILE, LANES), f32),
            pltpu.VMEM((NODES_PER_TILE, LANES), f32),
            pltpu.VMEM((NODES_PER_TILE, LANES), f32),
            pltpu.VMEM_SHARED((N_NODES, LANES), f32),
            pltpu.VMEM_SHARED((N_NODES, LANES), f32),
            pltpu.SemaphoreType.DMA,
            pltpu.SemaphoreType.DMA,
        ],
    )
    return kern(src2, dst2, x, w1p, b1p, w2p, b2p, ones)


BLK = 256


def _mlp_body(v_ref, w1_ref, b1_ref, w3_ref, b3_ref, o_ref, acc_ref):
    i = pl.program_id(0)
    u = jnp.dot(v_ref[...], w1_ref[...], preferred_element_type=jnp.float32)
    u = jnp.maximum(u + b1_ref[...], 0.0)
    p = jnp.dot(u, w3_ref[...], preferred_element_type=jnp.float32)

    @pl.when(i == 0)
    def _():
        acc_ref[...] = p

    @pl.when(i > 0)
    def _():
        acc_ref[...] = acc_ref[...] + p

    @pl.when(i == pl.num_programs(0) - 1)
    def _():
        z = acc_ref[...] + b3_ref[...]
        z = z - jnp.max(z, axis=-1, keepdims=True)
        e = jnp.exp(z)
        o_ref[...] = e / jnp.sum(e, axis=-1, keepdims=True)


def _mlp_tc(v, lin1_W, lin1_b, lin3_W, lin3_b):
    n_hidden = lin1_W.shape[1]
    n_out = lin3_W.shape[1]
    return pl.pallas_call(
        _mlp_body,
        grid=(n_hidden // BLK,),
        in_specs=[
            pl.BlockSpec((1, v.shape[1]), lambda i: (0, 0)),
            pl.BlockSpec((v.shape[1], BLK), lambda i: (0, i)),
            pl.BlockSpec((1, BLK), lambda i: (0, i)),
            pl.BlockSpec((BLK, n_out), lambda i: (i, 0)),
            pl.BlockSpec((1, n_out), lambda i: (0, 0)),
        ],
        out_specs=pl.BlockSpec((1, n_out), lambda i: (0, 0)),
        out_shape=jax.ShapeDtypeStruct((1, n_out), jnp.float32),
        scratch_shapes=[pltpu.VMEM((1, n_out), jnp.float32)],
    )(v, lin1_W, lin1_b, lin3_W, lin3_b)


def kernel(x, edge_index, W1, b1, W2, b2, lin1_W, lin1_b, lin3_W, lin3_b):
    f32 = jnp.float32
    src2 = edge_index[0].reshape(N_EDGES // CHUNK, CHUNK)
    dst2 = edge_index[1].reshape(N_EDGES // CHUNK, CHUNK)
    x16 = jnp.zeros((N_NODES, LANES), f32).at[:, :IN_DIM].set(x)
    w1p = jnp.zeros((IN_DIM, LANES), f32).at[:, :HID].set(W1)
    b1p = jnp.zeros((LANES,), f32).at[:HID].set(b1)
    w2p = jnp.zeros((HID, LANES), f32).at[:, :HID].set(W2)
    b2p = jnp.zeros((LANES,), f32).at[:HID].set(b2)
    ones = jnp.ones((CHUNK, LANES), f32)
    h2 = _gcn_sc(src2, dst2, x16, w1p, b1p, w2p, b2p, ones)
    v = h2[:, :HID].reshape(1, -1)
    out = _mlp_tc(v, lin1_W, lin1_b.reshape(1, -1), lin3_W, lin3_b.reshape(1, -1))
    return out.reshape(-1)
